# baseline (speedup 1.0000x reference)
import jax
import jax.numpy as jnp
from jax.experimental import pallas as pl
from jax.experimental.pallas import tpu as pltpu

B, S, H = 128, 577, 1024
E = 8
NEG_BIG = -3.0e38


def _router_kernel(vf_hbm, w_ref, b_ref, rw_ref, se_ref, cls_vmem, sem):
    cp = pltpu.make_async_copy(vf_hbm.at[:, 0:1, :], cls_vmem, sem)
    cp.start()
    cp.wait()
    cls = cls_vmem[:, 0, :]
    w = w_ref[...]
    logits = jax.lax.dot_general(
        cls, w, (((1,), (1,)), ((), ())),
        preferred_element_type=jnp.float32) + b_ref[...]
    idx = jax.lax.broadcasted_iota(jnp.int32, (B, E), 1)
    m1 = jnp.max(logits, axis=1, keepdims=True)
    i1 = jnp.min(jnp.where(logits == m1, idx, E), axis=1, keepdims=True)
    masked = jnp.where(idx == i1, NEG_BIG, logits)
    m2 = jnp.max(masked, axis=1, keepdims=True)
    i2 = jnp.min(jnp.where(masked == m2, idx, E), axis=1, keepdims=True)
    e = jnp.exp(m2 - m1)
    w1 = 1.0 / (1.0 + e)
    rw_ref[...] = jnp.concatenate([w1, 1.0 - w1], axis=1)
    se_ref[...] = jnp.concatenate([i1, i2], axis=1)


def kernel(vision_features, W, b):
    return pl.pallas_call(
        _router_kernel,
        out_shape=(
            jax.ShapeDtypeStruct((B, 2), jnp.float32),
            jax.ShapeDtypeStruct((B, 2), jnp.int32),
        ),
        in_specs=[
            pl.BlockSpec(memory_space=pl.ANY),
            pl.BlockSpec((E, H), lambda: (0, 0)),
            pl.BlockSpec((1, E), lambda: (0, 0)),
        ],
        out_specs=(
            pl.BlockSpec((B, 2), lambda: (0, 0)),
            pl.BlockSpec((B, 2), lambda: (0, 0)),
        ),
        scratch_shapes=[
            pltpu.VMEM((B, 1, H), jnp.float32),
            pltpu.SemaphoreType.DMA,
        ],
    )(vision_features, W, b.reshape(1, E))


# speedup vs baseline: 31.4886x; 31.4886x over previous
import jax
import jax.numpy as jnp
from jax.experimental import pallas as pl
from jax.experimental.pallas import tpu as pltpu

B, S, H = 128, 577, 1024
E = 8
NEG_BIG = -3.0e38


def _router_kernel(cls_ref, w_ref, b_ref, rw_ref, se_ref):
    cls = cls_ref[...]
    w = w_ref[...]
    logits = jax.lax.dot_general(
        cls, w, (((1,), (1,)), ((), ())),
        preferred_element_type=jnp.float32) + b_ref[...]
    idx = jax.lax.broadcasted_iota(jnp.int32, (B, E), 1)
    m1 = jnp.max(logits, axis=1, keepdims=True)
    i1 = jnp.min(jnp.where(logits == m1, idx, E), axis=1, keepdims=True)
    masked = jnp.where(idx == i1, NEG_BIG, logits)
    m2 = jnp.max(masked, axis=1, keepdims=True)
    i2 = jnp.min(jnp.where(masked == m2, idx, E), axis=1, keepdims=True)
    e = jnp.exp(m2 - m1)
    w1 = 1.0 / (1.0 + e)
    rw_ref[...] = jnp.concatenate([w1, 1.0 - w1], axis=1)
    se_ref[...] = jnp.concatenate([i1, i2], axis=1)


def kernel(vision_features, W, b):
    return pl.pallas_call(
        _router_kernel,
        out_shape=(
            jax.ShapeDtypeStruct((B, 2), jnp.float32),
            jax.ShapeDtypeStruct((B, 2), jnp.int32),
        ),
        in_specs=[
            pl.BlockSpec((B, H), lambda: (0, 0)),
            pl.BlockSpec((E, H), lambda: (0, 0)),
            pl.BlockSpec((1, E), lambda: (0, 0)),
        ],
        out_specs=(
            pl.BlockSpec((B, 2), lambda: (0, 0)),
            pl.BlockSpec((B, 2), lambda: (0, 0)),
        ),
    )(vision_features[:, 0, :], W, b.reshape(1, E))
